# baseline (speedup 1.0000x reference)
import dataclasses
import functools

import jax
import jax.numpy as jnp
from jax import lax
from jax.experimental import pallas as pl
from jax.experimental.pallas import tpu as pltpu
from jax.experimental.pallas import tpu_sc as plsc

_H = 1024
_F = 512
_E = 8
_K = 2
_T = 2048
_TB = 256
_NB = _T // _TB
_ROW_TILE = 256
_RT_LOG = 8
_NROWS_E = _T * _K + _E * _ROW_TILE
_NTILES_E = _NROWS_E // _ROW_TILE
_NTILES = _NTILES_E + _T // _ROW_TILE
_SHARED0 = _NROWS_E
_NROWS = _NROWS_E + _T


def _router_body(x_ref, gw_ref, w_ref, e_ref, rank_ref, base_ref, te_ref,
                 run_ref):
    i = pl.program_id(0)

    @pl.when(i == 0)
    def _():
        run_ref[...] = jnp.zeros_like(run_ref)

    x = x_ref[...]
    logits = lax.dot_general(gw_ref[...], x, (((1,), (1,)), ((), ())),
                             preferred_element_type=jnp.float32)
    scores = jax.nn.sigmoid(logits)
    siota = lax.broadcasted_iota(jnp.int32, scores.shape, 0)
    m1 = jnp.max(scores, axis=0, keepdims=True)
    i1 = jnp.min(jnp.where(scores == m1, siota, _E), axis=0, keepdims=True)
    masked = jnp.where(siota == i1, -jnp.inf, scores)
    m2 = jnp.max(masked, axis=0, keepdims=True)
    i2 = jnp.min(jnp.where(masked == m2, siota, _E), axis=0, keepdims=True)
    denom = m1 + m2 + 1e-20
    w_ref[...] = jnp.concatenate([m1 / denom, m2 / denom], axis=0)
    e_ref[...] = jnp.concatenate([i1, i2], axis=0)

    c = ((siota == i1) | (siota == i2)).astype(jnp.float32)
    tr = lax.broadcasted_iota(jnp.int32, (_TB, _TB), 0)
    tc = lax.broadcasted_iota(jnp.int32, (_TB, _TB), 1)
    u = (tr < tc).astype(jnp.bfloat16)
    p = lax.dot_general(c.astype(jnp.bfloat16), u, (((1,), (0,)), ((), ())),
                        preferred_element_type=jnp.float32)
    p_tot = p + run_ref[...]
    rank1 = jnp.sum(jnp.where(siota == i1, p_tot, 0.0), axis=0, keepdims=True)
    rank2 = jnp.sum(jnp.where(siota == i2, p_tot, 0.0), axis=0, keepdims=True)
    rank_ref[...] = jnp.concatenate([rank1, rank2], axis=0).astype(jnp.int32)

    run_ref[...] = run_ref[...] + jnp.sum(c, axis=1, keepdims=True)

    @pl.when(i == _NB - 1)
    def _():
        total = run_ref[...].astype(jnp.int32)
        pad = ((total + (_ROW_TILE - 1)) >> _RT_LOG) << _RT_LOG
        lr = lax.broadcasted_iota(jnp.int32, (_E, _E), 0)
        lc = lax.broadcasted_iota(jnp.int32, (_E, _E), 1)
        lmat = (lc < lr).astype(jnp.float32)
        base = lax.dot_general(lmat, pad.astype(jnp.float32),
                               (((1,), (0,)), ((), ())),
                               preferred_element_type=jnp.float32)
        er = lax.broadcasted_iota(jnp.int32, (_E, 16), 0)
        ec = lax.broadcasted_iota(jnp.int32, (_E, 16), 1)
        eye = (er == ec).astype(jnp.float32)
        base_lane = lax.dot_general(base, eye, (((0,), (0,)), ((), ())),
                                    preferred_element_type=jnp.float32)
        base_ref[...] = base_lane.astype(jnp.int32)

        used = jnp.sum(pad.astype(jnp.float32))
        jl = lax.broadcasted_iota(jnp.int32, (1, 64), 1)
        rowstart = (jl * _ROW_TILE).astype(jnp.float32)
        acc = jnp.zeros((1, 64), jnp.int32)
        for e in range(_E):
            be = jnp.sum(jnp.where(lax.broadcasted_iota(
                jnp.int32, (_E, 1), 0) == e, base, 0.0))
            acc = acc + (rowstart >= be).astype(jnp.int32)
        te = acc - 1
        te = jnp.where(rowstart >= used, -1, te)
        te = jnp.where(jl >= _NTILES_E,
                       jnp.where(jl < _NTILES, _E, -1), te)
        te_ref[...] = te[0]


def _router(x, gate_weight):
    return pl.pallas_call(
        _router_body,
        grid=(_NB,),
        in_specs=[
            pl.BlockSpec((_TB, _H), lambda i: (i, 0)),
            pl.BlockSpec((_E, _H), lambda i: (0, 0)),
        ],
        out_specs=(
            pl.BlockSpec((_K, _TB), lambda i: (0, i)),
            pl.BlockSpec((_K, _TB), lambda i: (0, i)),
            pl.BlockSpec((_K, _TB), lambda i: (0, i)),
            pl.BlockSpec((1, 16), lambda i: (0, 0)),
            pl.BlockSpec((64,), lambda i: (0,)),
        ),
        out_shape=(
            jax.ShapeDtypeStruct((_K, _T), jnp.float32),
            jax.ShapeDtypeStruct((_K, _T), jnp.int32),
            jax.ShapeDtypeStruct((_K, _T), jnp.int32),
            jax.ShapeDtypeStruct((1, 16), jnp.int32),
            jax.ShapeDtypeStruct((64,), jnp.int32),
        ),
        scratch_shapes=[pltpu.VMEM((_E, 1), jnp.float32)],
    )(x, gate_weight)


def _vmesh():
    return plsc.VectorSubcoreMesh(core_axis_name="c", subcore_axis_name="s")


def _sc_params():
    cp = pltpu.CompilerParams()
    if "needs_layout_passes" in pltpu.CompilerParams.__dataclass_fields__:
        cp = dataclasses.replace(cp, needs_layout_passes=False)
    return cp


def _dispatch_body(xb_ref, e_ref, rank_ref, base_ref,
                   xs_ref, pos_ref,
                   ev_v, rk_v, pos_v, rows_v, base_v):
    cid = lax.axis_index("c")
    sid = lax.axis_index("s")
    wid = sid * 2 + cid
    a0 = wid * 128
    k = a0 // _T
    t0 = a0 % _T

    pltpu.sync_copy(base_ref.at[0], base_v)
    pltpu.sync_copy(e_ref.at[k, pl.ds(t0, 128)], ev_v)
    pltpu.sync_copy(rank_ref.at[k, pl.ds(t0, 128)], rk_v)

    bvec = base_v[...]
    iota = lax.iota(jnp.int32, 16)
    bases = [jnp.sum(jnp.where(iota == e, bvec, 0)) for e in range(_E)]
    for c in range(8):
        ev = ev_v[pl.ds(c * 16, 16)]
        rk = rk_v[pl.ds(c * 16, 16)]
        badd = jnp.zeros((16,), jnp.int32)
        for e in range(_E):
            badd = jnp.where(ev == e, bases[e], badd)
        pos_v[pl.ds(c * 16, 16)] = rk + badd

    pltpu.sync_copy(pos_v, pos_ref.at[k, pl.ds(t0, 128)])
    pltpu.sync_copy(xb_ref.at[pl.ds(t0, 128)], rows_v)
    pltpu.sync_copy(rows_v, xs_ref.at[pos_v])


def _dispatch(xpk, e2, rank2, base16):
    f = pl.kernel(
        _dispatch_body,
        out_type=(
            jax.ShapeDtypeStruct((_NROWS_E, _H // 2), jnp.int32),
            jax.ShapeDtypeStruct((_K, _T), jnp.int32),
        ),
        mesh=_vmesh(),
        scratch_types=[
            pltpu.VMEM((128,), jnp.int32),
            pltpu.VMEM((128,), jnp.int32),
            pltpu.VMEM((128,), jnp.int32),
            pltpu.VMEM((128, _H // 2), jnp.int32),
            pltpu.VMEM((16,), jnp.int32),
        ],
        compiler_params=_sc_params(),
    )
    return f(xpk, e2, rank2, base16)


def _ffn_body(te_ref, xs_ref, xb_ref, wg_ref, wu_ref, wd_ref, eo_ref):
    j = pl.program_id(0)
    e = te_ref[j]

    @pl.when(e >= 0)
    def _():
        ee = jnp.maximum(e, 0)
        xi = jnp.where(j < _NTILES_E, xs_ref[...], xb_ref[...])
        xlo = lax.bitcast_convert_type(xi << 16, jnp.float32)
        xhi = lax.bitcast_convert_type(xi & jnp.int32(-65536), jnp.float32)
        xv = jnp.concatenate([xlo, xhi], axis=1).astype(jnp.bfloat16)
        h1 = jnp.dot(xv, wg_ref[ee], preferred_element_type=jnp.float32)
        g = h1 * jax.nn.sigmoid(h1)
        u = jnp.dot(xv, wu_ref[ee], preferred_element_type=jnp.float32)
        o = jnp.dot((g * u).astype(jnp.bfloat16), wd_ref[ee],
                    preferred_element_type=jnp.float32)
        olo = o[:, :_H // 2].astype(jnp.bfloat16).astype(jnp.float32)
        ohi = o[:, _H // 2:].astype(jnp.bfloat16).astype(jnp.float32)
        ilo = lax.bitcast_convert_type(olo, jnp.int32)
        ihi = lax.bitcast_convert_type(ohi, jnp.int32)
        eo_ref[...] = (ihi & jnp.int32(-65536)) | (
            (ilo >> 16) & jnp.int32(0xFFFF))


def _ffn(te, xs2, xb2, wg_all, wu_all, wd_all):
    grid_spec = pltpu.PrefetchScalarGridSpec(
        num_scalar_prefetch=1,
        grid=(_NTILES,),
        in_specs=[
            pl.BlockSpec((_ROW_TILE, _H // 2),
                         lambda j, te: (jnp.where(j < _NTILES_E, j, 0), 0)),
            pl.BlockSpec((_ROW_TILE, _H // 2),
                         lambda j, te: (jnp.where(j < _NTILES_E, 0,
                                                  j - _NTILES_E), 0)),
            pl.BlockSpec((_E + 1, _H, _F), lambda j, te: (0, 0, 0)),
            pl.BlockSpec((_E + 1, _H, _F), lambda j, te: (0, 0, 0)),
            pl.BlockSpec((_E + 1, _F, _H), lambda j, te: (0, 0, 0)),
        ],
        out_specs=pl.BlockSpec((_ROW_TILE, _H // 2), lambda j, te: (j, 0)),
    )
    return pl.pallas_call(
        _ffn_body,
        grid_spec=grid_spec,
        out_shape=jax.ShapeDtypeStruct((_NROWS, _H // 2), jnp.int32),
    )(te, xs2, xb2, wg_all, wu_all, wd_all)


_CT = 16


def _combine_body(eo_ref, w_ref, pos_ref, y_ref,
                  p0a, p1a, w0a, w1a, rsa, r0s, r1s, ys, gsems, ssems):
    cid = lax.axis_index("c")
    sid = lax.axis_index("s")
    wid = sid * 2 + cid
    tw = _T // 32
    t0 = wid * tw
    iota = lax.iota(jnp.int32, 16)
    himask = jnp.full((16,), -65536, jnp.int32)

    pltpu.sync_copy(pos_ref.at[0, pl.ds(t0, tw)], p0a)
    pltpu.sync_copy(pos_ref.at[1, pl.ds(t0, tw)], p1a)
    pltpu.sync_copy(w_ref.at[0, pl.ds(t0, tw)], w0a)
    pltpu.sync_copy(w_ref.at[1, pl.ds(t0, tw)], w1a)
    rs_d = pltpu.async_copy(eo_ref.at[pl.ds(_SHARED0 + t0, tw)], rsa,
                            gsems[0])

    def start(c):
        i = c % 2
        return [
            pltpu.async_copy(eo_ref.at[p0a.at[pl.ds(c * _CT, _CT)]],
                             r0s[i], gsems[i]),
            pltpu.async_copy(eo_ref.at[p1a.at[pl.ds(c * _CT, _CT)]],
                             r1s[i], gsems[i]),
        ]

    gathers = {0: start(0)}
    rs_d.wait()
    stores = {}
    for c in range(4):
        i = c % 2
        if c + 1 < 4:
            gathers[c + 1] = start(c + 1)
        for d in gathers[c]:
            d.wait()
        if c >= 2:
            stores[c - 2].wait()
        r0_v, r1_v, y_v = r0s[i], r1s[i], ys[i]
        w0v = w0a[pl.ds(c * _CT, 16)]
        w1v = w1a[pl.ds(c * _CT, 16)]

        @pl.loop(0, 16)
        def _(tok):
            a = jnp.sum(jnp.where(iota == tok, w0v, 0.0))
            b = jnp.sum(jnp.where(iota == tok, w1v, 0.0))
            srow = c * _CT + tok
            for g in range(_H // 32):
                sl = pl.ds(g * 16, 16)
                v0 = r0_v[tok, sl]
                v1 = r1_v[tok, sl]
                vs = rsa[srow, sl]
                lo0 = plsc.bitcast(v0 << 16, jnp.float32)
                lo1 = plsc.bitcast(v1 << 16, jnp.float32)
                los = plsc.bitcast(vs << 16, jnp.float32)
                hi0 = plsc.bitcast(v0 & himask, jnp.float32)
                hi1 = plsc.bitcast(v1 & himask, jnp.float32)
                his = plsc.bitcast(vs & himask, jnp.float32)
                y_v[tok, sl] = a * lo0 + b * lo1 + los
                y_v[tok, pl.ds(_H // 2 + g * 16, 16)] = (
                    a * hi0 + b * hi1 + his)

        stores[c] = pltpu.async_copy(
            y_v, y_ref.at[pl.ds(t0 + c * _CT, _CT)], ssems[i])
    stores[2].wait()
    stores[3].wait()


def _combine(eo2, w2, pos2):
    tw = _T // 32
    f = pl.kernel(
        _combine_body,
        out_type=jax.ShapeDtypeStruct((_T, _H), jnp.float32),
        mesh=_vmesh(),
        scratch_types=[
            pltpu.VMEM((tw,), jnp.int32),
            pltpu.VMEM((tw,), jnp.int32),
            pltpu.VMEM((tw,), jnp.float32),
            pltpu.VMEM((tw,), jnp.float32),
            pltpu.VMEM((tw, _H // 2), jnp.int32),
            [pltpu.VMEM((_CT, _H // 2), jnp.int32) for _ in range(2)],
            [pltpu.VMEM((_CT, _H // 2), jnp.int32) for _ in range(2)],
            [pltpu.VMEM((_CT, _H), jnp.float32) for _ in range(2)],
            [pltpu.SemaphoreType.DMA for _ in range(2)],
            [pltpu.SemaphoreType.DMA for _ in range(2)],
        ],
        compiler_params=_sc_params(),
    )
    return f(eo2, w2, pos2)


def kernel(hidden_states, gate_weight, w_gate, w_up, w_down,
           sw_gate, sw_up, sw_down):
    b, s, h = hidden_states.shape
    x = hidden_states.reshape(-1, h)
    bf = jnp.bfloat16

    xlo = x[:, :_H // 2].astype(bf).astype(jnp.float32)
    xhi = x[:, _H // 2:].astype(bf).astype(jnp.float32)
    xpk = (lax.bitcast_convert_type(xhi, jnp.int32) & jnp.int32(-65536)) | (
        (lax.bitcast_convert_type(xlo, jnp.int32) >> 16) & jnp.int32(0xFFFF))

    w2, e2, rank2, base16, te = _router(x, gate_weight)
    xs_pk, pos2 = _dispatch(xpk, e2, rank2, base16)

    wg_all = jnp.concatenate([w_gate, sw_gate[None]], axis=0).astype(bf)
    wu_all = jnp.concatenate([w_up, sw_up[None]], axis=0).astype(bf)
    wd_all = jnp.concatenate([w_down, sw_down[None]], axis=0).astype(bf)
    eo_pk = _ffn(te, xs_pk, xpk, wg_all, wu_all, wd_all)

    y = _combine(eo_pk, w2, pos2)
    return y.reshape(b, s, h)


# speedup vs baseline: 1.1190x; 1.1190x over previous
import dataclasses
import functools

import jax
import jax.numpy as jnp
from jax import lax
from jax.experimental import pallas as pl
from jax.experimental.pallas import tpu as pltpu
from jax.experimental.pallas import tpu_sc as plsc

_H = 1024
_F = 512
_E = 8
_K = 2
_T = 2048
_TB = 256
_NB = _T // _TB
_ROW_TILE = 256
_RT_LOG = 8
_NROWS_E = _T * _K + _E * _ROW_TILE
_NTILES_E = _NROWS_E // _ROW_TILE
_NTILES = _NTILES_E + _T // _ROW_TILE
_SHARED0 = _NROWS_E
_NROWS = _NROWS_E + _T


def _router_body(x_ref, gw_ref, w_ref, e_ref, rank_ref, base_ref, te_ref,
                 run_ref):
    i = pl.program_id(0)

    @pl.when(i == 0)
    def _():
        run_ref[...] = jnp.zeros_like(run_ref)

    x = x_ref[...]
    logits = lax.dot_general(gw_ref[...], x, (((1,), (1,)), ((), ())),
                             preferred_element_type=jnp.float32)
    scores = jax.nn.sigmoid(logits)
    siota = lax.broadcasted_iota(jnp.int32, scores.shape, 0)
    m1 = jnp.max(scores, axis=0, keepdims=True)
    i1 = jnp.min(jnp.where(scores == m1, siota, _E), axis=0, keepdims=True)
    masked = jnp.where(siota == i1, -jnp.inf, scores)
    m2 = jnp.max(masked, axis=0, keepdims=True)
    i2 = jnp.min(jnp.where(masked == m2, siota, _E), axis=0, keepdims=True)
    denom = m1 + m2 + 1e-20
    w_ref[...] = jnp.concatenate([m1 / denom, m2 / denom], axis=0)
    e_ref[...] = jnp.concatenate([i1, i2], axis=0)

    c = ((siota == i1) | (siota == i2)).astype(jnp.float32)
    tr = lax.broadcasted_iota(jnp.int32, (_TB, _TB), 0)
    tc = lax.broadcasted_iota(jnp.int32, (_TB, _TB), 1)
    u = (tr < tc).astype(jnp.bfloat16)
    p = lax.dot_general(c.astype(jnp.bfloat16), u, (((1,), (0,)), ((), ())),
                        preferred_element_type=jnp.float32)
    p_tot = p + run_ref[...]
    rank1 = jnp.sum(jnp.where(siota == i1, p_tot, 0.0), axis=0, keepdims=True)
    rank2 = jnp.sum(jnp.where(siota == i2, p_tot, 0.0), axis=0, keepdims=True)
    rank_ref[...] = jnp.concatenate([rank1, rank2], axis=0).astype(jnp.int32)

    run_ref[...] = run_ref[...] + jnp.sum(c, axis=1, keepdims=True)

    @pl.when(i == _NB - 1)
    def _():
        total = run_ref[...].astype(jnp.int32)
        pad = ((total + (_ROW_TILE - 1)) >> _RT_LOG) << _RT_LOG
        lr = lax.broadcasted_iota(jnp.int32, (_E, _E), 0)
        lc = lax.broadcasted_iota(jnp.int32, (_E, _E), 1)
        lmat = (lc < lr).astype(jnp.float32)
        base = lax.dot_general(lmat, pad.astype(jnp.float32),
                               (((1,), (0,)), ((), ())),
                               preferred_element_type=jnp.float32)
        er = lax.broadcasted_iota(jnp.int32, (_E, 16), 0)
        ec = lax.broadcasted_iota(jnp.int32, (_E, 16), 1)
        eye = (er == ec).astype(jnp.float32)
        base_lane = lax.dot_general(base, eye, (((0,), (0,)), ((), ())),
                                    preferred_element_type=jnp.float32)
        base_ref[...] = base_lane.astype(jnp.int32)

        used = jnp.sum(pad.astype(jnp.float32))
        jl = lax.broadcasted_iota(jnp.int32, (1, 64), 1)
        rowstart = (jl * _ROW_TILE).astype(jnp.float32)
        acc = jnp.zeros((1, 64), jnp.int32)
        for e in range(_E):
            be = jnp.sum(jnp.where(lax.broadcasted_iota(
                jnp.int32, (_E, 1), 0) == e, base, 0.0))
            acc = acc + (rowstart >= be).astype(jnp.int32)
        te = acc - 1
        te = jnp.where(rowstart >= used, -1, te)
        te = jnp.where(jl >= _NTILES_E, -1, te)
        te_ref[...] = te[0]


def _router(x, gate_weight):
    return pl.pallas_call(
        _router_body,
        grid=(_NB,),
        in_specs=[
            pl.BlockSpec((_TB, _H), lambda i: (i, 0)),
            pl.BlockSpec((_E, _H), lambda i: (0, 0)),
        ],
        out_specs=(
            pl.BlockSpec((_K, _TB), lambda i: (0, i)),
            pl.BlockSpec((_K, _TB), lambda i: (0, i)),
            pl.BlockSpec((_K, _TB), lambda i: (0, i)),
            pl.BlockSpec((1, 16), lambda i: (0, 0)),
            pl.BlockSpec((64,), lambda i: (0,)),
        ),
        out_shape=(
            jax.ShapeDtypeStruct((_K, _T), jnp.float32),
            jax.ShapeDtypeStruct((_K, _T), jnp.int32),
            jax.ShapeDtypeStruct((_K, _T), jnp.int32),
            jax.ShapeDtypeStruct((1, 16), jnp.int32),
            jax.ShapeDtypeStruct((64,), jnp.int32),
        ),
        scratch_shapes=[pltpu.VMEM((_E, 1), jnp.float32)],
    )(x, gate_weight)


def _vmesh():
    return plsc.VectorSubcoreMesh(core_axis_name="c", subcore_axis_name="s")


def _sc_params():
    cp = pltpu.CompilerParams()
    if "needs_layout_passes" in pltpu.CompilerParams.__dataclass_fields__:
        cp = dataclasses.replace(cp, needs_layout_passes=False)
    return cp


def _dispatch_body(xb_ref, e_ref, rank_ref, base_ref,
                   xs_ref, pos_ref,
                   ev_v, rk_v, pos_v, rows_v, base_v):
    cid = lax.axis_index("c")
    sid = lax.axis_index("s")
    wid = sid * 2 + cid
    a0 = wid * 128
    k = a0 // _T
    t0 = a0 % _T

    pltpu.sync_copy(base_ref.at[0], base_v)
    pltpu.sync_copy(e_ref.at[k, pl.ds(t0, 128)], ev_v)
    pltpu.sync_copy(rank_ref.at[k, pl.ds(t0, 128)], rk_v)

    bvec = base_v[...]
    iota = lax.iota(jnp.int32, 16)
    bases = [jnp.sum(jnp.where(iota == e, bvec, 0)) for e in range(_E)]
    for c in range(8):
        ev = ev_v[pl.ds(c * 16, 16)]
        rk = rk_v[pl.ds(c * 16, 16)]
        badd = jnp.zeros((16,), jnp.int32)
        for e in range(_E):
            badd = jnp.where(ev == e, bases[e], badd)
        pos_v[pl.ds(c * 16, 16)] = rk + badd

    pltpu.sync_copy(pos_v, pos_ref.at[k, pl.ds(t0, 128)])
    pltpu.sync_copy(xb_ref.at[pl.ds(t0, 128)], rows_v)
    pltpu.sync_copy(rows_v, xs_ref.at[pos_v])


def _dispatch(xpk, e2, rank2, base16):
    f = pl.kernel(
        _dispatch_body,
        out_type=(
            jax.ShapeDtypeStruct((_NROWS_E, _H // 2), jnp.int32),
            jax.ShapeDtypeStruct((_K, _T), jnp.int32),
        ),
        mesh=_vmesh(),
        scratch_types=[
            pltpu.VMEM((128,), jnp.int32),
            pltpu.VMEM((128,), jnp.int32),
            pltpu.VMEM((128,), jnp.int32),
            pltpu.VMEM((128, _H // 2), jnp.int32),
            pltpu.VMEM((16,), jnp.int32),
        ],
        compiler_params=_sc_params(),
    )
    return f(xpk, e2, rank2, base16)


def _shared_body(x_ref, swg_ref, swu_ref, swd_ref, out_ref):
    xb = x_ref[...].astype(jnp.bfloat16)
    h1 = jnp.dot(xb, swg_ref[...], preferred_element_type=jnp.float32)
    g = h1 * jax.nn.sigmoid(h1)
    u = jnp.dot(xb, swu_ref[...], preferred_element_type=jnp.float32)
    out_ref[...] = jnp.dot((g * u).astype(jnp.bfloat16), swd_ref[...],
                           preferred_element_type=jnp.float32)


def _shared(x, swg, swu, swd):
    return pl.pallas_call(
        _shared_body,
        grid=(_NB,),
        in_specs=[
            pl.BlockSpec((_TB, _H), lambda i: (i, 0)),
            pl.BlockSpec((_H, _F), lambda i: (0, 0)),
            pl.BlockSpec((_H, _F), lambda i: (0, 0)),
            pl.BlockSpec((_F, _H), lambda i: (0, 0)),
        ],
        out_specs=pl.BlockSpec((_TB, _H), lambda i: (i, 0)),
        out_shape=jax.ShapeDtypeStruct((_T, _H), jnp.float32),
    )(x, swg, swu, swd)


def _ffn_body(te_ref, xs_ref, wg_ref, wu_ref, wd_ref, eo_ref):
    j = pl.program_id(0)
    e = te_ref[j]

    @pl.when(e >= 0)
    def _():
        ee = jnp.maximum(e, 0)
        xi = xs_ref[...]
        xlo = lax.bitcast_convert_type(xi << 16, jnp.float32)
        xhi = lax.bitcast_convert_type(xi & jnp.int32(-65536), jnp.float32)
        xv = jnp.concatenate([xlo, xhi], axis=1).astype(jnp.bfloat16)
        h1 = jnp.dot(xv, wg_ref[ee], preferred_element_type=jnp.float32)
        g = h1 * jax.nn.sigmoid(h1)
        u = jnp.dot(xv, wu_ref[ee], preferred_element_type=jnp.float32)
        o = jnp.dot((g * u).astype(jnp.bfloat16), wd_ref[ee],
                    preferred_element_type=jnp.float32)
        olo = o[:, :_H // 2].astype(jnp.bfloat16).astype(jnp.float32)
        ohi = o[:, _H // 2:].astype(jnp.bfloat16).astype(jnp.float32)
        ilo = lax.bitcast_convert_type(olo, jnp.int32)
        ihi = lax.bitcast_convert_type(ohi, jnp.int32)
        eo_ref[...] = (ihi & jnp.int32(-65536)) | (
            (ilo >> 16) & jnp.int32(0xFFFF))


def _ffn(te, xs2, wg_all, wu_all, wd_all):
    grid_spec = pltpu.PrefetchScalarGridSpec(
        num_scalar_prefetch=1,
        grid=(_NTILES_E,),
        in_specs=[
            pl.BlockSpec((_ROW_TILE, _H // 2), lambda j, te: (j, 0)),
            pl.BlockSpec((_E, _H, _F), lambda j, te: (0, 0, 0)),
            pl.BlockSpec((_E, _H, _F), lambda j, te: (0, 0, 0)),
            pl.BlockSpec((_E, _F, _H), lambda j, te: (0, 0, 0)),
        ],
        out_specs=pl.BlockSpec((_ROW_TILE, _H // 2), lambda j, te: (j, 0)),
    )
    return pl.pallas_call(
        _ffn_body,
        grid_spec=grid_spec,
        out_shape=jax.ShapeDtypeStruct((_NROWS_E, _H // 2), jnp.int32),
    )(te, xs2, wg_all, wu_all, wd_all)


_CT = 16


def _combine_body(eo_ref, sh_ref, w_ref, pos_ref, y_ref,
                  p0a, p1a, w0a, w1a, r0s, r1s, rss, ys, gsems, ssems):
    cid = lax.axis_index("c")
    sid = lax.axis_index("s")
    wid = sid * 2 + cid
    tw = _T // 32
    t0 = wid * tw
    iota = lax.iota(jnp.int32, 16)
    himask = jnp.full((16,), -65536, jnp.int32)

    pltpu.sync_copy(pos_ref.at[0, pl.ds(t0, tw)], p0a)
    pltpu.sync_copy(pos_ref.at[1, pl.ds(t0, tw)], p1a)
    pltpu.sync_copy(w_ref.at[0, pl.ds(t0, tw)], w0a)
    pltpu.sync_copy(w_ref.at[1, pl.ds(t0, tw)], w1a)

    def start(c):
        i = c % 2
        return [
            pltpu.async_copy(eo_ref.at[p0a.at[pl.ds(c * _CT, _CT)]],
                             r0s[i], gsems[i]),
            pltpu.async_copy(eo_ref.at[p1a.at[pl.ds(c * _CT, _CT)]],
                             r1s[i], gsems[i]),
            pltpu.async_copy(sh_ref.at[pl.ds(t0 + c * _CT, _CT)],
                             rss[i], gsems[i]),
        ]

    gathers = {0: start(0)}
    stores = {}
    for c in range(4):
        i = c % 2
        if c + 1 < 4:
            gathers[c + 1] = start(c + 1)
        for d in gathers[c]:
            d.wait()
        if c >= 2:
            stores[c - 2].wait()
        r0_v, r1_v, rs_v, y_v = r0s[i], r1s[i], rss[i], ys[i]
        w0v = w0a[pl.ds(c * _CT, 16)]
        w1v = w1a[pl.ds(c * _CT, 16)]

        @pl.loop(0, 16)
        def _(tok):
            a = jnp.sum(jnp.where(iota == tok, w0v, 0.0))
            b = jnp.sum(jnp.where(iota == tok, w1v, 0.0))
            for g in range(_H // 32):
                sl = pl.ds(g * 16, 16)
                sl2 = pl.ds(_H // 2 + g * 16, 16)
                v0 = r0_v[tok, sl]
                v1 = r1_v[tok, sl]
                lo0 = plsc.bitcast(v0 << 16, jnp.float32)
                lo1 = plsc.bitcast(v1 << 16, jnp.float32)
                hi0 = plsc.bitcast(v0 & himask, jnp.float32)
                hi1 = plsc.bitcast(v1 & himask, jnp.float32)
                y_v[tok, sl] = a * lo0 + b * lo1 + rs_v[tok, sl]
                y_v[tok, sl2] = a * hi0 + b * hi1 + rs_v[tok, sl2]

        stores[c] = pltpu.async_copy(
            y_v, y_ref.at[pl.ds(t0 + c * _CT, _CT)], ssems[i])
    stores[2].wait()
    stores[3].wait()


def _combine(eo2, sh, w2, pos2):
    tw = _T // 32
    f = pl.kernel(
        _combine_body,
        out_type=jax.ShapeDtypeStruct((_T, _H), jnp.float32),
        mesh=_vmesh(),
        scratch_types=[
            pltpu.VMEM((tw,), jnp.int32),
            pltpu.VMEM((tw,), jnp.int32),
            pltpu.VMEM((tw,), jnp.float32),
            pltpu.VMEM((tw,), jnp.float32),
            [pltpu.VMEM((_CT, _H // 2), jnp.int32) for _ in range(2)],
            [pltpu.VMEM((_CT, _H // 2), jnp.int32) for _ in range(2)],
            [pltpu.VMEM((_CT, _H), jnp.float32) for _ in range(2)],
            [pltpu.VMEM((_CT, _H), jnp.float32) for _ in range(2)],
            [pltpu.SemaphoreType.DMA for _ in range(2)],
            [pltpu.SemaphoreType.DMA for _ in range(2)],
        ],
        compiler_params=_sc_params(),
    )
    return f(eo2, sh, w2, pos2)


def kernel(hidden_states, gate_weight, w_gate, w_up, w_down,
           sw_gate, sw_up, sw_down):
    b, s, h = hidden_states.shape
    x = hidden_states.reshape(-1, h)
    bf = jnp.bfloat16

    xlo = x[:, :_H // 2].astype(bf).astype(jnp.float32)
    xhi = x[:, _H // 2:].astype(bf).astype(jnp.float32)
    xpk = (lax.bitcast_convert_type(xhi, jnp.int32) & jnp.int32(-65536)) | (
        (lax.bitcast_convert_type(xlo, jnp.int32) >> 16) & jnp.int32(0xFFFF))

    w2, e2, rank2, base16, te = _router(x, gate_weight)
    xs_pk, pos2 = _dispatch(xpk, e2, rank2, base16)

    shared = _shared(x, sw_gate.astype(bf), sw_up.astype(bf),
                     sw_down.astype(bf))

    eo_pk = _ffn(te, xs_pk, w_gate.astype(bf), w_up.astype(bf),
                 w_down.astype(bf))

    y = _combine(eo_pk, shared, w2, pos2)
    return y.reshape(b, s, h)
